# baseline (speedup 1.0000x reference)
import functools

import jax
import jax.numpy as jnp
from jax import lax
from jax.experimental import pallas as pl
from jax.experimental.pallas import tpu as pltpu
from jax.experimental.pallas import tpu_sc as plsc

N = 10000
E = 320000
D_IN = 128
H = 256
D_OUT = 10
G = 64

N_PAD = 10240
K = 128
NC = 2
NS = 16
HH = H // 2
NCHUNK = 2560
E_PAD = NCHUNK * K
CH_T = NCHUNK // NS
CH_W = NCHUNK // (NC * NS)
BCH = 40

KS = 64
DEPTH = 5
CH_T2 = E_PAD // KS // NS
BCHS = 40

_HIGH = lax.Precision.DEFAULT


def _deg_body(dst2d_hbm, deg_hbm, acc_sh, didx, ones_v, zeros_v, dsem):
    c = lax.axis_index("c")
    s = lax.axis_index("s")
    wid = c * NS + s

    @pl.loop(0, K, step=16)
    def _(j):
        ones_v[pl.ds(j, 16)] = jnp.full((16,), 1.0, jnp.float32)
        zeros_v[pl.ds(j, 16)] = jnp.zeros((16,), jnp.float32)

    @pl.loop(0, 5)
    def _(i):
        pltpu.sync_copy(zeros_v, acc_sh.at[pl.ds(s * 640 + i * K, K)])

    pltpu.sync_copy(dst2d_hbm.at[pl.ds(wid * CH_W, CH_W)], didx)

    plsc.subcore_barrier()

    @pl.loop(0, CH_W, step=8)
    def _(i):
        for k in range(8):
            pltpu.async_copy(ones_v, acc_sh.at[didx.at[i + k]], dsem, add=True)
        for k in range(8):
            pltpu.make_async_copy(ones_v, acc_sh.at[didx.at[i + k]], dsem).wait()

    plsc.subcore_barrier()
    pltpu.sync_copy(acc_sh.at[pl.ds(s * 640, 640)],
                    deg_hbm.at[c, pl.ds(s * 640, 640)])


def _agg_body(lo_hbm, hi_hbm, src_hbm, dst2d_hbm, out_lo, out_hi,
              acc_sh, sidx, didx, rows, *sems):
    c = lax.axis_index("c")
    s = lax.axis_index("s")

    @pl.loop(0, DEPTH * KS)
    def _(r):
        @pl.loop(0, HH, step=16)
        def _(j):
            rows[r, pl.ds(j, 16)] = jnp.zeros((16,), jnp.float32)

    @pl.loop(0, 5)
    def _(i):
        pltpu.sync_copy(rows.at[pl.ds(0, 128)],
                        acc_sh.at[pl.ds(s * 640 + i * 128, 128)])

    plsc.subcore_barrier()

    def _slot(k):
        return rows.at[pl.ds(k * KS, KS)]

    def _gather(slot, i, sem):
        idx = sidx.at[pl.ds(i * KS, KS)]

        @pl.when(c == 0)
        def _():
            pltpu.async_copy(lo_hbm.at[idx], _slot(slot), sem)

        @pl.when(c == 1)
        def _():
            pltpu.async_copy(hi_hbm.at[idx], _slot(slot), sem)

    def _gwait(slot, i, sem):
        idx = sidx.at[pl.ds(i * KS, KS)]

        @pl.when(c == 0)
        def _():
            pltpu.make_async_copy(lo_hbm.at[idx], _slot(slot), sem).wait()

        @pl.when(c == 1)
        def _():
            pltpu.make_async_copy(hi_hbm.at[idx], _slot(slot), sem).wait()

    def _scatter(slot, i):
        pltpu.sync_copy(_slot(slot), acc_sh.at[didx.at[i]], add=True)

    @pl.loop(0, CH_T2 // BCHS)
    def _(b):
        c0 = s * CH_T2 + b * BCHS
        pltpu.sync_copy(src_hbm.at[pl.ds(c0 * KS, BCHS * KS)], sidx)
        pltpu.sync_copy(dst2d_hbm.at[pl.ds(c0, BCHS)], didx)
        for k in range(DEPTH):
            _gather(k, k, sems[k])

        @pl.loop(0, BCHS, step=DEPTH)
        def _(i):
            for k in range(DEPTH):
                _gwait(k, i + k, sems[k])
                _scatter(k, i + k)

                @pl.when(i + k + DEPTH < BCHS)
                def _():
                    _gather(k, i + k + DEPTH, sems[k])

    plsc.subcore_barrier()

    @pl.when(c == 0)
    def _():
        pltpu.sync_copy(acc_sh.at[pl.ds(s * 640, 640)],
                        out_lo.at[pl.ds(s * 640, 640)])

    @pl.when(c == 1)
    def _():
        pltpu.sync_copy(acc_sh.at[pl.ds(s * 640, 640)],
                        out_hi.at[pl.ds(s * 640, 640)])


@functools.cache
def _sc_kernels():
    mesh = plsc.VectorSubcoreMesh(core_axis_name="c", subcore_axis_name="s")
    deg_sc = pl.kernel(
        _deg_body,
        out_type=jax.ShapeDtypeStruct((NC, N_PAD), jnp.float32),
        mesh=mesh,
        scratch_types=[
            pltpu.VMEM_SHARED((N_PAD,), jnp.float32),
            pltpu.VMEM((CH_W, K), jnp.int32),
            pltpu.VMEM((K,), jnp.float32),
            pltpu.VMEM((K,), jnp.float32),
            pltpu.SemaphoreType.DMA,
        ],
    )
    agg_sc = pl.kernel(
        _agg_body,
        out_type=[jax.ShapeDtypeStruct((N_PAD, HH), jnp.float32),
                  jax.ShapeDtypeStruct((N_PAD, HH), jnp.float32)],
        mesh=mesh,
        scratch_types=[
            pltpu.VMEM_SHARED((N_PAD, HH), jnp.float32),
            pltpu.VMEM((BCHS * KS,), jnp.int32),
            pltpu.VMEM((BCHS, KS), jnp.int32),
            pltpu.VMEM((DEPTH * KS, HH), jnp.float32),
        ] + [pltpu.SemaphoreType.DMA] * DEPTH,
    )
    return deg_sc, agg_sc


def _mm1_body(x_ref, w_ref, d0_ref, d1_ref, lo_ref, hi_ref, dinv_ref):
    deg = d0_ref[...] + d1_ref[...] + 1.0
    dinv = lax.rsqrt(deg)
    hp = jnp.dot(x_ref[...], w_ref[...],
                 preferred_element_type=jnp.float32, precision=_HIGH) * dinv
    lo_ref[...] = hp[:, :HH]
    hi_ref[...] = hp[:, HH:]
    dinv_ref[...] = dinv


def _layer2_body(alo_ref, ahi_ref, lo_ref, hi_ref, dinv_ref, b1_ref, w2_ref,
                 olo_ref, ohi_ref):
    dinv = dinv_ref[...]
    h1 = jnp.concatenate(
        [alo_ref[...] + lo_ref[...], ahi_ref[...] + hi_ref[...]], axis=1)
    h1 = jnp.maximum(h1 * dinv + b1_ref[...], 0.0)
    hp2 = jnp.dot(h1, w2_ref[...],
                  preferred_element_type=jnp.float32, precision=_HIGH) * dinv
    olo_ref[...] = hp2[:, :HH]
    ohi_ref[...] = hp2[:, HH:]


def _final_body(alo_ref, ahi_ref, lo_ref, hi_ref, dinv_ref, b2_ref,
                batch_ref, wc_ref, bc_ref, o_ref):
    dinv = dinv_ref[...]
    h2 = jnp.concatenate(
        [alo_ref[...] + lo_ref[...], ahi_ref[...] + hi_ref[...]], axis=1)
    h2 = jnp.maximum(h2 * dinv + b2_ref[...], 0.0)
    pt = (lax.broadcasted_iota(jnp.int32, (G, N_PAD), 0)
          == batch_ref[...]).astype(jnp.float32)
    sums = jnp.dot(pt, h2, preferred_element_type=jnp.float32, precision=_HIGH)
    counts = jnp.sum(pt, axis=1, keepdims=True)
    pooled = sums / jnp.maximum(counts, 1.0)
    o_ref[...] = jnp.dot(pooled, wc_ref[...],
                         preferred_element_type=jnp.float32,
                         precision=_HIGH) + bc_ref[...]


def _mm1(x, W1, d0, d1):
    nb = 10
    rb = N_PAD // nb
    row = lambda i: (i, 0)
    full = lambda i: (0, 0)
    return pl.pallas_call(
        _mm1_body,
        grid=(nb,),
        in_specs=[pl.BlockSpec((rb, D_IN), row),
                  pl.BlockSpec((D_IN, H), full),
                  pl.BlockSpec((rb, 1), row),
                  pl.BlockSpec((rb, 1), row)],
        out_specs=[pl.BlockSpec((rb, HH), row),
                   pl.BlockSpec((rb, HH), row),
                   pl.BlockSpec((rb, 1), row)],
        out_shape=[jax.ShapeDtypeStruct((N_PAD, HH), jnp.float32),
                   jax.ShapeDtypeStruct((N_PAD, HH), jnp.float32),
                   jax.ShapeDtypeStruct((N_PAD, 1), jnp.float32)])(x, W1, d0, d1)


def _layer2(alo, ahi, lo, hi, dinv, b1r, W2):
    nb = 10
    rb = N_PAD // nb
    row = lambda i: (i, 0)
    full = lambda i: (0, 0)
    return pl.pallas_call(
        _layer2_body,
        grid=(nb,),
        in_specs=[pl.BlockSpec((rb, HH), row),
                  pl.BlockSpec((rb, HH), row),
                  pl.BlockSpec((rb, HH), row),
                  pl.BlockSpec((rb, HH), row),
                  pl.BlockSpec((rb, 1), row),
                  pl.BlockSpec((1, H), full),
                  pl.BlockSpec((H, H), full)],
        out_specs=[pl.BlockSpec((rb, HH), row),
                   pl.BlockSpec((rb, HH), row)],
        out_shape=[jax.ShapeDtypeStruct((N_PAD, HH), jnp.float32),
                   jax.ShapeDtypeStruct((N_PAD, HH), jnp.float32)],
    )(alo, ahi, lo, hi, dinv, b1r, W2)


def _final(alo, ahi, lo, hi, dinv, b2r, batch_row, Wc, bcr):
    return pl.pallas_call(
        _final_body,
        out_shape=jax.ShapeDtypeStruct((G, D_OUT), jnp.float32),
    )(alo, ahi, lo, hi, dinv, b2r, batch_row, Wc, bcr)


def kernel(x, edge_index, batch, W1, b1, W2, b2, Wc, bc):
    fill = (jnp.arange(E_PAD - E, dtype=jnp.int32) % (N_PAD - N))
    src = jnp.concatenate([edge_index[0], fill])
    dst_p = jnp.concatenate([edge_index[1], N + fill])
    dst2d = dst_p.reshape(NCHUNK, K)
    dst2ds = dst_p.reshape(E_PAD // KS, KS)
    x_pad = jnp.concatenate(
        [x, jnp.zeros((N_PAD - N, D_IN), jnp.float32)])
    batch_pad = jnp.concatenate(
        [batch, jnp.full((N_PAD - N,), G, jnp.int32)]).reshape(1, N_PAD)
    _deg_sc, _agg_sc = _sc_kernels()

    degp = _deg_sc(dst2d)
    lo1, hi1, dinv = _mm1(x_pad, W1, degp[0, :, None], degp[1, :, None])

    alo1, ahi1 = _agg_sc(lo1, hi1, src, dst2ds)
    lo2, hi2 = _layer2(alo1, ahi1, lo1, hi1, dinv,
                       b1.reshape(1, H), W2)

    alo2, ahi2 = _agg_sc(lo2, hi2, src, dst2ds)
    return _final(alo2, ahi2, lo2, hi2, dinv,
                  b2.reshape(1, H), batch_pad, Wc,
                  bc.reshape(1, D_OUT))


# speedup vs baseline: 1.0607x; 1.0607x over previous
import functools

import jax
import jax.numpy as jnp
from jax import lax
from jax.experimental import pallas as pl
from jax.experimental.pallas import tpu as pltpu
from jax.experimental.pallas import tpu_sc as plsc

N = 10000
E = 320000
D_IN = 128
H = 256
D_OUT = 10
G = 64

N_PAD = 10240
K = 128
NC = 2
NS = 16
HH = H // 2
NCHUNK = 2560
E_PAD = NCHUNK * K
CH_T = NCHUNK // NS
CH_W = NCHUNK // (NC * NS)
BCH = 40

KS = 64
DEPTH = 4
CH_T2 = E_PAD // KS // NS
BCHS = 80

_HIGH = lax.Precision.DEFAULT


def _deg_body(dst2d_hbm, deg_hbm, acc_sh, didx, ones_v, zeros_v, dsem):
    c = lax.axis_index("c")
    s = lax.axis_index("s")
    wid = c * NS + s

    @pl.loop(0, K, step=16)
    def _(j):
        ones_v[pl.ds(j, 16)] = jnp.full((16,), 1.0, jnp.float32)
        zeros_v[pl.ds(j, 16)] = jnp.zeros((16,), jnp.float32)

    @pl.loop(0, 5)
    def _(i):
        pltpu.sync_copy(zeros_v, acc_sh.at[pl.ds(s * 640 + i * K, K)])

    pltpu.sync_copy(dst2d_hbm.at[pl.ds(wid * CH_W, CH_W)], didx)

    plsc.subcore_barrier()

    @pl.loop(0, CH_W, step=8)
    def _(i):
        for k in range(8):
            pltpu.async_copy(ones_v, acc_sh.at[didx.at[i + k]], dsem, add=True)
        for k in range(8):
            pltpu.make_async_copy(ones_v, acc_sh.at[didx.at[i + k]], dsem).wait()

    plsc.subcore_barrier()
    pltpu.sync_copy(acc_sh.at[pl.ds(s * 640, 640)],
                    deg_hbm.at[c, pl.ds(s * 640, 640)])


def _agg_body(lo_hbm, hi_hbm, src_hbm, dst2d_hbm, out_lo, out_hi,
              acc_sh, sidx, didx, rows, *sems):
    c = lax.axis_index("c")
    s = lax.axis_index("s")

    @pl.loop(0, DEPTH * KS)
    def _(r):
        @pl.loop(0, HH, step=16)
        def _(j):
            rows[r, pl.ds(j, 16)] = jnp.zeros((16,), jnp.float32)

    @pl.loop(0, 5)
    def _(i):
        pltpu.sync_copy(rows.at[pl.ds(0, 128)],
                        acc_sh.at[pl.ds(s * 640 + i * 128, 128)])

    plsc.subcore_barrier()

    def _slot(k):
        return rows.at[pl.ds(k * KS, KS)]

    def _gather(slot, i, sem):
        idx = sidx.at[pl.ds(i * KS, KS)]

        @pl.when(c == 0)
        def _():
            pltpu.async_copy(lo_hbm.at[idx], _slot(slot), sem)

        @pl.when(c == 1)
        def _():
            pltpu.async_copy(hi_hbm.at[idx], _slot(slot), sem)

    def _gwait(slot, i, sem):
        idx = sidx.at[pl.ds(i * KS, KS)]

        @pl.when(c == 0)
        def _():
            pltpu.make_async_copy(lo_hbm.at[idx], _slot(slot), sem).wait()

        @pl.when(c == 1)
        def _():
            pltpu.make_async_copy(hi_hbm.at[idx], _slot(slot), sem).wait()

    def _scatter(slot, i):
        pltpu.sync_copy(_slot(slot), acc_sh.at[didx.at[i]], add=True)

    @pl.loop(0, CH_T2 // BCHS)
    def _(b):
        c0 = s * CH_T2 + b * BCHS
        pltpu.sync_copy(src_hbm.at[pl.ds(c0 * KS, BCHS * KS)], sidx)
        pltpu.sync_copy(dst2d_hbm.at[pl.ds(c0, BCHS)], didx)
        for k in range(DEPTH):
            _gather(k, k, sems[k])

        @pl.loop(0, BCHS, step=DEPTH)
        def _(i):
            for k in range(DEPTH):
                _gwait(k, i + k, sems[k])
                _scatter(k, i + k)

                @pl.when(i + k + DEPTH < BCHS)
                def _():
                    _gather(k, i + k + DEPTH, sems[k])

    plsc.subcore_barrier()

    @pl.when(c == 0)
    def _():
        pltpu.sync_copy(acc_sh.at[pl.ds(s * 640, 640)],
                        out_lo.at[pl.ds(s * 640, 640)])

    @pl.when(c == 1)
    def _():
        pltpu.sync_copy(acc_sh.at[pl.ds(s * 640, 640)],
                        out_hi.at[pl.ds(s * 640, 640)])


@functools.cache
def _sc_kernels():
    mesh = plsc.VectorSubcoreMesh(core_axis_name="c", subcore_axis_name="s")
    deg_sc = pl.kernel(
        _deg_body,
        out_type=jax.ShapeDtypeStruct((NC, N_PAD), jnp.float32),
        mesh=mesh,
        scratch_types=[
            pltpu.VMEM_SHARED((N_PAD,), jnp.float32),
            pltpu.VMEM((CH_W, K), jnp.int32),
            pltpu.VMEM((K,), jnp.float32),
            pltpu.VMEM((K,), jnp.float32),
            pltpu.SemaphoreType.DMA,
        ],
    )
    agg_sc = pl.kernel(
        _agg_body,
        out_type=[jax.ShapeDtypeStruct((N_PAD, HH), jnp.float32),
                  jax.ShapeDtypeStruct((N_PAD, HH), jnp.float32)],
        mesh=mesh,
        scratch_types=[
            pltpu.VMEM_SHARED((N_PAD, HH), jnp.float32),
            pltpu.VMEM((BCHS * KS,), jnp.int32),
            pltpu.VMEM((BCHS, KS), jnp.int32),
            pltpu.VMEM((DEPTH * KS, HH), jnp.float32),
        ] + [pltpu.SemaphoreType.DMA] * DEPTH,
    )
    return deg_sc, agg_sc


def _mm1_body(x_ref, w_ref, d0_ref, d1_ref, lo_ref, hi_ref, dinv_ref):
    deg = d0_ref[...] + d1_ref[...] + 1.0
    dinv = lax.rsqrt(deg)
    hp = jnp.dot(x_ref[...], w_ref[...],
                 preferred_element_type=jnp.float32, precision=_HIGH) * dinv
    lo_ref[...] = hp[:, :HH]
    hi_ref[...] = hp[:, HH:]
    dinv_ref[...] = dinv


def _layer2_body(alo_ref, ahi_ref, lo_ref, hi_ref, dinv_ref, b1_ref, w2_ref,
                 olo_ref, ohi_ref):
    dinv = dinv_ref[...]
    h1 = jnp.concatenate(
        [alo_ref[...] + lo_ref[...], ahi_ref[...] + hi_ref[...]], axis=1)
    h1 = jnp.maximum(h1 * dinv + b1_ref[...], 0.0)
    hp2 = jnp.dot(h1, w2_ref[...],
                  preferred_element_type=jnp.float32, precision=_HIGH) * dinv
    olo_ref[...] = hp2[:, :HH]
    ohi_ref[...] = hp2[:, HH:]


def _final_body(alo_ref, ahi_ref, lo_ref, hi_ref, dinv_ref, b2_ref,
                batch_ref, wc_ref, bc_ref, o_ref):
    dinv = dinv_ref[...]
    h2 = jnp.concatenate(
        [alo_ref[...] + lo_ref[...], ahi_ref[...] + hi_ref[...]], axis=1)
    h2 = jnp.maximum(h2 * dinv + b2_ref[...], 0.0)
    pt = (lax.broadcasted_iota(jnp.int32, (G, N_PAD), 0)
          == batch_ref[...]).astype(jnp.float32)
    sums = jnp.dot(pt, h2, preferred_element_type=jnp.float32, precision=_HIGH)
    counts = jnp.sum(pt, axis=1, keepdims=True)
    pooled = sums / jnp.maximum(counts, 1.0)
    o_ref[...] = jnp.dot(pooled, wc_ref[...],
                         preferred_element_type=jnp.float32,
                         precision=_HIGH) + bc_ref[...]


def _mm1(x, W1, d0, d1):
    nb = 10
    rb = N_PAD // nb
    row = lambda i: (i, 0)
    full = lambda i: (0, 0)
    return pl.pallas_call(
        _mm1_body,
        grid=(nb,),
        in_specs=[pl.BlockSpec((rb, D_IN), row),
                  pl.BlockSpec((D_IN, H), full),
                  pl.BlockSpec((rb, 1), row),
                  pl.BlockSpec((rb, 1), row)],
        out_specs=[pl.BlockSpec((rb, HH), row),
                   pl.BlockSpec((rb, HH), row),
                   pl.BlockSpec((rb, 1), row)],
        out_shape=[jax.ShapeDtypeStruct((N_PAD, HH), jnp.float32),
                   jax.ShapeDtypeStruct((N_PAD, HH), jnp.float32),
                   jax.ShapeDtypeStruct((N_PAD, 1), jnp.float32)])(x, W1, d0, d1)


def _layer2(alo, ahi, lo, hi, dinv, b1r, W2):
    nb = 10
    rb = N_PAD // nb
    row = lambda i: (i, 0)
    full = lambda i: (0, 0)
    return pl.pallas_call(
        _layer2_body,
        grid=(nb,),
        in_specs=[pl.BlockSpec((rb, HH), row),
                  pl.BlockSpec((rb, HH), row),
                  pl.BlockSpec((rb, HH), row),
                  pl.BlockSpec((rb, HH), row),
                  pl.BlockSpec((rb, 1), row),
                  pl.BlockSpec((1, H), full),
                  pl.BlockSpec((H, H), full)],
        out_specs=[pl.BlockSpec((rb, HH), row),
                   pl.BlockSpec((rb, HH), row)],
        out_shape=[jax.ShapeDtypeStruct((N_PAD, HH), jnp.float32),
                   jax.ShapeDtypeStruct((N_PAD, HH), jnp.float32)],
    )(alo, ahi, lo, hi, dinv, b1r, W2)


def _final(alo, ahi, lo, hi, dinv, b2r, batch_row, Wc, bcr):
    return pl.pallas_call(
        _final_body,
        out_shape=jax.ShapeDtypeStruct((G, D_OUT), jnp.float32),
    )(alo, ahi, lo, hi, dinv, b2r, batch_row, Wc, bcr)


def kernel(x, edge_index, batch, W1, b1, W2, b2, Wc, bc):
    fill = (jnp.arange(E_PAD - E, dtype=jnp.int32) % (N_PAD - N))
    src = jnp.concatenate([edge_index[0], fill])
    dst_p = jnp.concatenate([edge_index[1], N + fill])
    dst2d = dst_p.reshape(NCHUNK, K)
    dst2ds = dst_p.reshape(E_PAD // KS, KS)
    x_pad = jnp.concatenate(
        [x, jnp.zeros((N_PAD - N, D_IN), jnp.float32)])
    batch_pad = jnp.concatenate(
        [batch, jnp.full((N_PAD - N,), G, jnp.int32)]).reshape(1, N_PAD)
    _deg_sc, _agg_sc = _sc_kernels()

    degp = _deg_sc(dst2d)
    lo1, hi1, dinv = _mm1(x_pad, W1, degp[0, :, None], degp[1, :, None])

    alo1, ahi1 = _agg_sc(lo1, hi1, src, dst2ds)
    lo2, hi2 = _layer2(alo1, ahi1, lo1, hi1, dinv,
                       b1.reshape(1, H), W2)

    alo2, ahi2 = _agg_sc(lo2, hi2, src, dst2ds)
    return _final(alo2, ahi2, lo2, hi2, dinv,
                  b2.reshape(1, H), batch_pad, Wc,
                  bc.reshape(1, D_OUT))


# speedup vs baseline: 1.0616x; 1.0009x over previous
import functools

import jax
import jax.numpy as jnp
from jax import lax
from jax.experimental import pallas as pl
from jax.experimental.pallas import tpu as pltpu
from jax.experimental.pallas import tpu_sc as plsc

N = 10000
E = 320000
D_IN = 128
H = 256
D_OUT = 10
G = 64

N_PAD = 10240
K = 128
NC = 2
NS = 16
HH = H // 2
NCHUNK = 2560
E_PAD = NCHUNK * K
CH_W = NCHUNK // (NC * NS)

KS = 64
DEPTH = 4
CH_T2 = E_PAD // KS // NS
BCHS = 80

_HIGH = lax.Precision.DEFAULT


def _deg_body(dst2d_hbm, deg_hbm, acc_sh, didx, ones_v, zeros_v, dsem):
    c = lax.axis_index("c")
    s = lax.axis_index("s")
    wid = c * NS + s

    @pl.loop(0, K, step=16)
    def _(j):
        ones_v[pl.ds(j, 16)] = jnp.full((16,), 1.0, jnp.float32)
        zeros_v[pl.ds(j, 16)] = jnp.zeros((16,), jnp.float32)

    @pl.loop(0, 5)
    def _(i):
        pltpu.sync_copy(zeros_v, acc_sh.at[pl.ds(s * 640 + i * K, K)])

    pltpu.sync_copy(dst2d_hbm.at[pl.ds(wid * CH_W, CH_W)], didx)

    plsc.subcore_barrier()

    @pl.loop(0, CH_W, step=8)
    def _(i):
        for k in range(8):
            pltpu.async_copy(ones_v, acc_sh.at[didx.at[i + k]], dsem, add=True)
        for k in range(8):
            pltpu.make_async_copy(ones_v, acc_sh.at[didx.at[i + k]], dsem).wait()

    plsc.subcore_barrier()
    pltpu.sync_copy(acc_sh.at[pl.ds(s * 640, 640)],
                    deg_hbm.at[c, pl.ds(s * 640, 640)])


def _agg_body(lo_hbm, hi_hbm, src_hbm, dst2d_hbm, out_lo, out_hi,
              acc_sh, sidx, didx, rows, *sems):
    c = lax.axis_index("c")
    s = lax.axis_index("s")

    @pl.loop(0, DEPTH * KS)
    def _(r):
        @pl.loop(0, HH, step=16)
        def _(j):
            rows[r, pl.ds(j, 16)] = jnp.zeros((16,), jnp.float32)

    @pl.loop(0, 5)
    def _(i):
        pltpu.sync_copy(rows.at[pl.ds(0, 128)],
                        acc_sh.at[pl.ds(s * 640 + i * 128, 128)])

    plsc.subcore_barrier()

    def _slot(k):
        return rows.at[pl.ds(k * KS, KS)]

    def _gather(slot, i, sem):
        idx = sidx.at[pl.ds(i * KS, KS)]

        @pl.when(c == 0)
        def _():
            pltpu.async_copy(lo_hbm.at[idx], _slot(slot), sem)

        @pl.when(c == 1)
        def _():
            pltpu.async_copy(hi_hbm.at[idx], _slot(slot), sem)

    def _gwait(slot, i, sem):
        idx = sidx.at[pl.ds(i * KS, KS)]

        @pl.when(c == 0)
        def _():
            pltpu.make_async_copy(lo_hbm.at[idx], _slot(slot), sem).wait()

        @pl.when(c == 1)
        def _():
            pltpu.make_async_copy(hi_hbm.at[idx], _slot(slot), sem).wait()

    def _scatter(slot, i):
        pltpu.sync_copy(_slot(slot), acc_sh.at[didx.at[i]], add=True)

    @pl.loop(0, CH_T2 // BCHS)
    def _(b):
        c0 = s * CH_T2 + b * BCHS
        pltpu.sync_copy(src_hbm.at[pl.ds(c0 * KS, BCHS * KS)], sidx)
        pltpu.sync_copy(dst2d_hbm.at[pl.ds(c0, BCHS)], didx)
        for k in range(DEPTH):
            _gather(k, k, sems[k])

        @pl.loop(0, BCHS, step=DEPTH)
        def _(i):
            for k in range(DEPTH):
                _gwait(k, i + k, sems[k])
                _scatter(k, i + k)

                @pl.when(i + k + DEPTH < BCHS)
                def _():
                    _gather(k, i + k + DEPTH, sems[k])

    plsc.subcore_barrier()

    @pl.when(c == 0)
    def _():
        pltpu.sync_copy(acc_sh.at[pl.ds(s * 640, 640)],
                        out_lo.at[pl.ds(s * 640, 640)])

    @pl.when(c == 1)
    def _():
        pltpu.sync_copy(acc_sh.at[pl.ds(s * 640, 640)],
                        out_hi.at[pl.ds(s * 640, 640)])


@functools.cache
def _sc_kernels():
    mesh = plsc.VectorSubcoreMesh(core_axis_name="c", subcore_axis_name="s")
    deg_sc = pl.kernel(
        _deg_body,
        out_type=jax.ShapeDtypeStruct((NC, N_PAD), jnp.float32),
        mesh=mesh,
        scratch_types=[
            pltpu.VMEM_SHARED((N_PAD,), jnp.float32),
            pltpu.VMEM((CH_W, K), jnp.int32),
            pltpu.VMEM((K,), jnp.float32),
            pltpu.VMEM((K,), jnp.float32),
            pltpu.SemaphoreType.DMA,
        ],
    )
    agg_sc = pl.kernel(
        _agg_body,
        out_type=[jax.ShapeDtypeStruct((N_PAD, HH), jnp.float32),
                  jax.ShapeDtypeStruct((N_PAD, HH), jnp.float32)],
        mesh=mesh,
        scratch_types=[
            pltpu.VMEM_SHARED((N_PAD, HH), jnp.float32),
            pltpu.VMEM((BCHS * KS,), jnp.int32),
            pltpu.VMEM((BCHS, KS), jnp.int32),
            pltpu.VMEM((DEPTH * KS, HH), jnp.float32),
        ] + [pltpu.SemaphoreType.DMA] * DEPTH,
    )
    return deg_sc, agg_sc


def _mm1_body(x_ref, w_ref, d0_ref, d1_ref, lo_ref, hi_ref, dinv_ref):
    deg = d0_ref[...] + d1_ref[...] + 1.0
    dinv = lax.rsqrt(deg)
    hp = jnp.dot(x_ref[...], w_ref[...],
                 preferred_element_type=jnp.float32, precision=_HIGH) * dinv
    lo_ref[...] = hp[:, :HH]
    hi_ref[...] = hp[:, HH:]
    dinv_ref[...] = dinv


def _layer2_body(alo_ref, ahi_ref, lo_ref, hi_ref, dinv_ref, b1_ref, w2_ref,
                 olo_ref, ohi_ref):
    dinv = dinv_ref[...]
    h1 = jnp.concatenate(
        [alo_ref[...] + lo_ref[...], ahi_ref[...] + hi_ref[...]], axis=1)
    h1 = jnp.maximum(h1 * dinv + b1_ref[...], 0.0)
    hp2 = jnp.dot(h1, w2_ref[...],
                  preferred_element_type=jnp.float32, precision=_HIGH) * dinv
    olo_ref[...] = hp2[:, :HH]
    ohi_ref[...] = hp2[:, HH:]


def _final_body(alo_ref, ahi_ref, lo_ref, hi_ref, dinv_ref, b2_ref,
                batch_ref, wc_ref, bc_ref, o_ref):
    dinv = dinv_ref[...]
    h2 = jnp.concatenate(
        [alo_ref[...] + lo_ref[...], ahi_ref[...] + hi_ref[...]], axis=1)
    h2 = jnp.maximum(h2 * dinv + b2_ref[...], 0.0)
    pt = (lax.broadcasted_iota(jnp.int32, (G, N_PAD), 0)
          == batch_ref[...]).astype(jnp.float32)
    sums = jnp.dot(pt, h2, preferred_element_type=jnp.float32, precision=_HIGH)
    counts = jnp.sum(pt, axis=1, keepdims=True)
    pooled = sums / jnp.maximum(counts, 1.0)
    o_ref[...] = jnp.dot(pooled, wc_ref[...],
                         preferred_element_type=jnp.float32,
                         precision=_HIGH) + bc_ref[...]


def _mm1(x, W1, d0, d1):
    nb = 10
    rb = N_PAD // nb
    row = lambda i: (i, 0)
    full = lambda i: (0, 0)
    return pl.pallas_call(
        _mm1_body,
        grid=(nb,),
        in_specs=[pl.BlockSpec((rb, D_IN), row),
                  pl.BlockSpec((D_IN, H), full),
                  pl.BlockSpec((rb, 1), row),
                  pl.BlockSpec((rb, 1), row)],
        out_specs=[pl.BlockSpec((rb, HH), row),
                   pl.BlockSpec((rb, HH), row),
                   pl.BlockSpec((rb, 1), row)],
        out_shape=[jax.ShapeDtypeStruct((N_PAD, HH), jnp.float32),
                   jax.ShapeDtypeStruct((N_PAD, HH), jnp.float32),
                   jax.ShapeDtypeStruct((N_PAD, 1), jnp.float32)])(x, W1, d0, d1)


def _layer2(alo, ahi, lo, hi, dinv, b1r, W2):
    nb = 10
    rb = N_PAD // nb
    row = lambda i: (i, 0)
    full = lambda i: (0, 0)
    return pl.pallas_call(
        _layer2_body,
        grid=(nb,),
        in_specs=[pl.BlockSpec((rb, HH), row),
                  pl.BlockSpec((rb, HH), row),
                  pl.BlockSpec((rb, HH), row),
                  pl.BlockSpec((rb, HH), row),
                  pl.BlockSpec((rb, 1), row),
                  pl.BlockSpec((1, H), full),
                  pl.BlockSpec((H, H), full)],
        out_specs=[pl.BlockSpec((rb, HH), row),
                   pl.BlockSpec((rb, HH), row)],
        out_shape=[jax.ShapeDtypeStruct((N_PAD, HH), jnp.float32),
                   jax.ShapeDtypeStruct((N_PAD, HH), jnp.float32)],
    )(alo, ahi, lo, hi, dinv, b1r, W2)


def _final(alo, ahi, lo, hi, dinv, b2r, batch_row, Wc, bcr):
    return pl.pallas_call(
        _final_body,
        out_shape=jax.ShapeDtypeStruct((G, D_OUT), jnp.float32),
    )(alo, ahi, lo, hi, dinv, b2r, batch_row, Wc, bcr)


def kernel(x, edge_index, batch, W1, b1, W2, b2, Wc, bc):
    fill = (jnp.arange(E_PAD - E, dtype=jnp.int32) % (N_PAD - N))
    src = jnp.concatenate([edge_index[0], fill])
    dst_p = jnp.concatenate([edge_index[1], N + fill])
    dst2d = dst_p.reshape(NCHUNK, K)
    dst2ds = dst_p.reshape(E_PAD // KS, KS)
    x_pad = jnp.concatenate(
        [x, jnp.zeros((N_PAD - N, D_IN), jnp.float32)])
    batch_pad = jnp.concatenate(
        [batch, jnp.full((N_PAD - N,), G, jnp.int32)]).reshape(1, N_PAD)
    _deg_sc, _agg_sc = _sc_kernels()

    degp = _deg_sc(dst2d)
    lo1, hi1, dinv = _mm1(x_pad, W1, degp[0, :, None], degp[1, :, None])

    alo1, ahi1 = _agg_sc(lo1, hi1, src, dst2ds)
    lo2, hi2 = _layer2(alo1, ahi1, lo1, hi1, dinv,
                       b1.reshape(1, H), W2)

    alo2, ahi2 = _agg_sc(lo2, hi2, src, dst2ds)
    return _final(alo2, ahi2, lo2, hi2, dinv,
                  b2.reshape(1, H), batch_pad, Wc,
                  bc.reshape(1, D_OUT))
